# baseline (speedup 1.0000x reference)
import functools

import jax
import jax.numpy as jnp
from jax import lax
from jax.experimental import pallas as pl
from jax.experimental.pallas import tpu as pltpu
from jax.experimental.pallas import tpu_sc as plsc

EMB = 300
NROW = 25
LANES = 16
NPAD = 32
FULL_CHUNKS = 18
TAIL_OFF = EMB - LANES
WPAD = FULL_CHUNKS * LANES + LANES
BLK = 128


def _make_sc_call():
    mesh = plsc.VectorSubcoreMesh(core_axis_name="c", subcore_axis_name="s")

    @functools.partial(
        pl.kernel,
        out_type=jax.ShapeDtypeStruct((NPAD,), jnp.float32),
        mesh=mesh,
        compiler_params=pltpu.CompilerParams(
            needs_layout_passes=False, use_tc_tiling_on_sc=True,
            skip_device_barrier=True),
        scratch_types=[
            pltpu.VMEM((NPAD,), jnp.int32),
            pltpu.VMEM((WPAD,), jnp.float32),
            pltpu.VMEM((LANES,), jnp.float32),
            pltpu.VMEM((EMB, BLK), jnp.float32),
            pltpu.VMEM((LANES,), jnp.float32),
            pltpu.VMEM((LANES * LANES,), jnp.float32),
            pltpu.VMEM((LANES,), jnp.float32),
            pltpu.VMEM_SHARED((LANES * LANES,), jnp.float32),
            pltpu.SemaphoreType.DMA,
        ],
    )
    def sc_fn(idx_hbm, wp_hbm, b_hbm, tableT_hbm, out_hbm,
              idx_v, w_v, b_v, blk_v, stage_v, flat_v, out_v, acc_sh, sem):
        cid = lax.axis_index("c")
        sid = lax.axis_index("s")
        row = cid * LANES + sid

        pltpu.sync_copy(idx_hbm, idx_v)
        pltpu.sync_copy(wp_hbm, w_v)

        lane = lax.iota(jnp.int32, LANES)
        r = plsc.load_gather(idx_v, [jnp.full((LANES,), row, jnp.int32)])[0]

        tb = pl.multiple_of((r // BLK) * BLK, BLK)
        cp = pltpu.async_copy(tableT_hbm.at[:, pl.ds(tb, BLK)], blk_v, sem)
        col = r - tb

        wchunks = [w_v[pl.ds(c * LANES, LANES)] for c in range(FULL_CHUNKS)]
        wtail = w_v[pl.ds(FULL_CHUNKS * LANES, LANES)]

        cp.wait()
        colv = jnp.full((LANES,), col, jnp.int32)
        acc = plsc.load_gather(blk_v, [lane, colv]) * wchunks[0]
        for c in range(1, FULL_CHUNKS):
            acc = acc + plsc.load_gather(
                blk_v, [c * LANES + lane, colv]) * wchunks[c]
        acc = acc + plsc.load_gather(blk_v, [TAIL_OFF + lane, colv]) * wtail

        stage_v[pl.ds(0, LANES)] = acc
        off = pl.multiple_of(sid * LANES, LANES)
        pltpu.sync_copy(stage_v, acc_sh.at[pl.ds(off, LANES)])
        plsc.subcore_barrier()

        @pl.when(sid == 0)
        def _():
            pltpu.sync_copy(b_hbm, b_v)
            pltpu.sync_copy(acc_sh, flat_v)
            bias = b_v[...]
            base = lane * LANES
            tot = plsc.load_gather(flat_v, [base])
            for j in range(1, LANES):
                tot = tot + plsc.load_gather(flat_v, [base + j])
            x = tot + bias
            out_v[pl.ds(0, LANES)] = 1.0 / (1.0 + jnp.exp(-x))
            oof = pl.multiple_of(cid * LANES, LANES)
            pltpu.sync_copy(out_v, out_hbm.at[pl.ds(oof, LANES)])

    return sc_fn


_SC_CALL = _make_sc_call()


def kernel(input_words, table, W, b):
    idx = jnp.zeros((NPAD,), jnp.int32).at[:NROW].set(input_words[:, -1])
    w0 = W[:, 0]
    wp = jnp.concatenate(
        [w0[: FULL_CHUNKS * LANES], jnp.zeros((4,), jnp.float32), w0[FULL_CHUNKS * LANES:]]
    )
    bvec = jnp.full((LANES,), b[0], jnp.float32)
    out = _SC_CALL(idx, wp, bvec, jnp.swapaxes(table, 0, 1))
    return out[:NROW]


# speedup vs baseline: 1.0975x; 1.0975x over previous
import functools

import jax
import jax.numpy as jnp
from jax import lax
from jax.experimental import pallas as pl
from jax.experimental.pallas import tpu as pltpu
from jax.experimental.pallas import tpu_sc as plsc

EMB = 300
NROW = 25
LANES = 16
NPAD = 32
FULL_CHUNKS = 18
TAIL_OFF = EMB - LANES
WPAD = FULL_CHUNKS * LANES + LANES
BLK = 128


def _make_sc_call():
    mesh = plsc.VectorSubcoreMesh(core_axis_name="c", subcore_axis_name="s")

    @functools.partial(
        pl.kernel,
        out_type=jax.ShapeDtypeStruct((NPAD,), jnp.float32),
        mesh=mesh,
        compiler_params=pltpu.CompilerParams(
            needs_layout_passes=False, use_tc_tiling_on_sc=True,
            skip_device_barrier=True),
        scratch_types=[
            pltpu.VMEM((NPAD,), jnp.int32),
            pltpu.VMEM((1, EMB), jnp.float32),
            pltpu.VMEM((1,), jnp.float32),
            pltpu.VMEM((EMB, BLK), jnp.float32),
            pltpu.VMEM((LANES,), jnp.float32),
            pltpu.VMEM((LANES * LANES,), jnp.float32),
            pltpu.VMEM((LANES,), jnp.float32),
            pltpu.VMEM_SHARED((LANES * LANES,), jnp.float32),
            pltpu.SemaphoreType.DMA,
        ],
    )
    def sc_fn(idx_hbm, wT_hbm, b_hbm, tableT_hbm, out_hbm,
              idx_v, w_v, b_v, blk_v, stage_v, flat_v, out_v, acc_sh, sem):
        cid = lax.axis_index("c")
        sid = lax.axis_index("s")
        row = cid * LANES + sid

        pltpu.sync_copy(idx_hbm, idx_v)
        pltpu.sync_copy(wT_hbm, w_v)

        lane = lax.iota(jnp.int32, LANES)
        r = plsc.load_gather(idx_v, [jnp.full((LANES,), row, jnp.int32)])[0]

        tb = pl.multiple_of((r // BLK) * BLK, BLK)
        cp = pltpu.async_copy(tableT_hbm.at[:, pl.ds(tb, BLK)], blk_v, sem)
        col = r - tb

        wchunks = [w_v[0, pl.ds(c * LANES, LANES)] for c in range(FULL_CHUNKS)]
        wtail = jnp.where(lane < (LANES - (EMB - FULL_CHUNKS * LANES)),
                          jnp.zeros((LANES,), jnp.float32),
                          w_v[0, pl.ds(TAIL_OFF, LANES)])

        cp.wait()
        colv = jnp.full((LANES,), col, jnp.int32)
        acc = plsc.load_gather(blk_v, [lane, colv]) * wchunks[0]
        for c in range(1, FULL_CHUNKS):
            acc = acc + plsc.load_gather(
                blk_v, [c * LANES + lane, colv]) * wchunks[c]
        acc = acc + plsc.load_gather(blk_v, [TAIL_OFF + lane, colv]) * wtail

        stage_v[pl.ds(0, LANES)] = acc
        off = pl.multiple_of(sid * LANES, LANES)
        pltpu.sync_copy(stage_v, acc_sh.at[pl.ds(off, LANES)])
        plsc.subcore_barrier()

        @pl.when(sid == 0)
        def _():
            pltpu.sync_copy(b_hbm, b_v)
            pltpu.sync_copy(acc_sh, flat_v)
            bias = plsc.load_gather(b_v, [jnp.zeros((LANES,), jnp.int32)])
            base = lane * LANES
            tot = plsc.load_gather(flat_v, [base])
            for j in range(1, LANES):
                tot = tot + plsc.load_gather(flat_v, [base + j])
            x = tot + bias
            out_v[pl.ds(0, LANES)] = 1.0 / (1.0 + jnp.exp(-x))
            oof = pl.multiple_of(cid * LANES, LANES)
            pltpu.sync_copy(out_v, out_hbm.at[pl.ds(oof, LANES)])

    return sc_fn


_SC_CALL = _make_sc_call()


def kernel(input_words, table, W, b):
    idx = jnp.pad(input_words[:, -1], (0, NPAD - NROW))
    out = _SC_CALL(idx, jnp.swapaxes(W, 0, 1), b, jnp.swapaxes(table, 0, 1))
    return out[:NROW]
